# baseline (speedup 1.0000x reference)
import functools

import jax
import jax.numpy as jnp
from jax import lax
from jax.experimental import pallas as pl
from jax.experimental.pallas import tpu as pltpu
from jax.experimental.pallas import tpu_sc as plsc

_NTJ = 25
_NTI = 128
_NP = 200


@functools.cache
def _make_kernel():
    info = plsc.get_sparse_core_info()
    nc, ns = info.num_cores, info.num_subcores
    nw = nc * ns
    tpw = _NTI // nw

    mesh = plsc.VectorSubcoreMesh(core_axis_name="c", subcore_axis_name="s")

    @functools.partial(
        pl.kernel,
        mesh=mesh,
        out_type=jax.ShapeDtypeStruct((200 * 4 * 128 * 1024,), jnp.float32),
        compiler_params=pltpu.CompilerParams(
            use_tc_tiling_on_sc=False, needs_layout_passes=False
        ),
        scratch_types=(
            [pltpu.VMEM((128,), jnp.int32) for _ in range(2)]
            + [pltpu.VMEM((128, 32), jnp.float32) for _ in range(2)]
            + [pltpu.VMEM((4096,), jnp.float32) for _ in range(2)]
            + [pltpu.SemaphoreType.DMA for _ in range(6)]
        ),
    )
    def gather_kernel(idx_hbm, table_hbm, out_hbm, *scratch):
        idx_q = scratch[0:2]
        rows = scratch[2:4]
        tiles = scratch[4:6]
        sem_i = scratch[6:8]
        sem_g = scratch[8:10]
        sem_o = scratch[10:12]

        wid = lax.axis_index("s") * nc + lax.axis_index("c")
        iota = lax.iota(jnp.int32, 16)

        def idx_cp(p, ti, k2):
            off = (p >> 3) * (128 * 1024) + ti * 1024 + (p & 7) * 128
            return pltpu.make_async_copy(
                idx_hbm.at[pl.ds(off, 128)], idx_q[k2], sem_i[k2]
            )

        def gather_cp(k2):
            return pltpu.make_async_copy(
                table_hbm.at[idx_q[k2]], rows[k2], sem_g[k2]
            )

        def out_cps(j, ti, k2):
            return [
                pltpu.make_async_copy(
                    tiles[k2].at[pl.ds(tr * 1024, 1024)],
                    out_hbm.at[pl.ds(((j * 4 + tr) * 128 + ti) * 1024, 1024)],
                    sem_o[k2],
                )
                for tr in range(4)
            ]

        for t in range(tpw):
            ti = wid * tpw + t

            idx_cp(0, ti, 0).start()
            idx_cp(1, ti, 1).start()
            idx_cp(0, ti, 0).wait()
            gather_cp(0).start()
            idx_cp(1, ti, 1).wait()
            gather_cp(1).start()

            def body(q2, carry):
                for k2 in range(2):
                    p = 2 * q2 + k2
                    gather_cp(k2).wait()

                    @pl.when(p + 2 < _NP)
                    def _():
                        idx_cp(p + 2, ti, k2).start()

                    @pl.when(q2 >= 1)
                    def _():
                        for cp in out_cps(p, ti, k2):
                            cp.wait()

                    for c in range(128):
                        v0 = rows[k2][c, pl.ds(0, 16)]
                        v1 = rows[k2][c, pl.ds(16, 16)]
                        plsc.store_scatter(tiles[k2], [iota * 128 + c], v0)
                        plsc.store_scatter(
                            tiles[k2], [iota * 128 + (2048 + c)], v1
                        )

                    for cp in out_cps(p, ti, k2):
                        cp.start()

                    @pl.when(p + 2 < _NP)
                    def _():
                        idx_cp(p + 2, ti, k2).wait()
                        gather_cp(k2).start()
                return carry

            lax.fori_loop(0, _NP // 2, body, 0)

            for k2 in range(2):
                for cp in out_cps(k2, ti, k2):
                    cp.wait()

    return gather_kernel


def kernel(token_ids, weight):
    t4 = token_ids.T.reshape(_NTJ, 8, _NTI, 128)
    idx_view = t4.transpose(0, 2, 1, 3).reshape(-1)
    x = _make_kernel()(idx_view, weight)
    y = x.reshape(200, 4, 128, 8, 128).transpose(2, 4, 0, 1, 3)
    return y.reshape(16384, 200, 32)


# speedup vs baseline: 1.8392x; 1.8392x over previous
import functools

import jax
import jax.numpy as jnp
from jax import lax
from jax.experimental import pallas as pl
from jax.experimental.pallas import tpu as pltpu
from jax.experimental.pallas import tpu_sc as plsc

_NTJ = 25
_NTI = 128
_NP = 200


@functools.cache
def _make_kernel():
    info = plsc.get_sparse_core_info()
    nc, ns = info.num_cores, info.num_subcores
    nw = nc * ns
    tpw = _NTI // nw

    mesh = plsc.VectorSubcoreMesh(core_axis_name="c", subcore_axis_name="s")

    @functools.partial(
        pl.kernel,
        mesh=mesh,
        out_type=jax.ShapeDtypeStruct((200, 4, 128, 8, 128), jnp.float32),
        compiler_params=pltpu.CompilerParams(
            use_tc_tiling_on_sc=False, needs_layout_passes=False
        ),
        scratch_types=(
            [pltpu.VMEM((128,), jnp.int32) for _ in range(2)]
            + [pltpu.VMEM((128, 32), jnp.float32) for _ in range(2)]
            + [pltpu.VMEM((32, 129), jnp.float32) for _ in range(2)]
            + [pltpu.SemaphoreType.DMA for _ in range(6)]
        ),
    )
    def gather_kernel(idx_hbm, table_hbm, out_hbm, *scratch):
        idx_q = scratch[0:2]
        rows = scratch[2:4]
        tiles = scratch[4:6]
        sem_i = scratch[6:8]
        sem_g = scratch[8:10]
        sem_o = scratch[10:12]

        wid = lax.axis_index("s") * nc + lax.axis_index("c")
        iota = lax.iota(jnp.int32, 16)

        def idx_cp(p, ti, k2):
            off = (p >> 3) * (128 * 1024) + ti * 1024 + (p & 7) * 128
            return pltpu.make_async_copy(
                idx_hbm.at[pl.ds(off, 128)], idx_q[k2], sem_i[k2]
            )

        def gather_cp(k2):
            return pltpu.make_async_copy(
                table_hbm.at[idx_q[k2]], rows[k2], sem_g[k2]
            )

        def out_cps(j, ti, k2):
            return [
                pltpu.make_async_copy(
                    tiles[k2].at[pl.ds(8 * tr, 8), pl.ds(0, 128)],
                    out_hbm.at[j, tr, ti],
                    sem_o[k2],
                )
                for tr in range(4)
            ]

        for t in range(tpw):
            ti = wid * tpw + t

            idx_cp(0, ti, 0).start()
            idx_cp(1, ti, 1).start()
            idx_cp(0, ti, 0).wait()
            gather_cp(0).start()
            idx_cp(1, ti, 1).wait()
            gather_cp(1).start()

            def body(q2, carry):
                for k2 in range(2):
                    p = 2 * q2 + k2
                    gather_cp(k2).wait()

                    @pl.when(p + 2 < _NP)
                    def _():
                        idx_cp(p + 2, ti, k2).start()

                    @pl.when(q2 >= 1)
                    def _():
                        for cp in out_cps(p, ti, k2):
                            cp.wait()

                    for c in range(128):
                        v0 = rows[k2][c, pl.ds(0, 16)]
                        v1 = rows[k2][c, pl.ds(16, 16)]
                        cvec = jnp.full((16,), c, jnp.int32)
                        plsc.store_scatter(tiles[k2], [iota, cvec], v0)
                        plsc.store_scatter(tiles[k2], [iota + 16, cvec], v1)

                    for cp in out_cps(p, ti, k2):
                        cp.start()

                    @pl.when(p + 2 < _NP)
                    def _():
                        idx_cp(p + 2, ti, k2).wait()
                        gather_cp(k2).start()
                return carry

            lax.fori_loop(0, _NP // 2, body, 0)

            for k2 in range(2):
                for cp in out_cps(k2, ti, k2):
                    cp.wait()

    return gather_kernel


def kernel(token_ids, weight):
    t4 = token_ids.T.reshape(_NTJ, 8, _NTI, 128)
    idx_view = t4.transpose(0, 2, 1, 3).reshape(-1)
    x = _make_kernel()(idx_view, weight)
    y = x.transpose(2, 4, 0, 1, 3)
    return y.reshape(16384, 200, 32)
